# Initial kernel scaffold; baseline (speedup 1.0000x reference)
import jax
import jax.numpy as jnp
from jax.experimental import pallas as pl


def kernel(labels, edge_index, emb, W11, b11, W12, b12, W21, b21, W22, b22, Ws1, bs1, Ws2, bs2):
    raise NotImplementedError("write your pallas kernel here")


# speedup vs baseline: 2.4610x; 2.4610x over previous
import functools

import jax
import jax.numpy as jnp
from jax import lax
from jax.experimental import pallas as pl
from jax.experimental.pallas import tpu as pltpu
from jax.experimental.pallas import tpu_sc as plsc

_N = 50000
_NP = 51200
_E = 800000
_NC = 2
_NS = 16
_BN = 800
_NB = _NP // _BN
_EPT = _E // _NS
_EB = 80
_NBATCH = _EPT // _EB
_RPT = _NP // _NS
_ZR = 320


def _embed_call(lab3, embp):
    def body(lab_ref, emb_ref, out_ref):
        half = pl.program_id(0)
        lab = jnp.clip(lab_ref[0, 0, :], 0, 50)
        oh = (lab[:, None] == lax.broadcasted_iota(jnp.int32, (_BN, 64), 1))
        h = jnp.dot(oh.astype(jnp.float32), emb_ref[...],
                    preferred_element_type=jnp.float32,
                    precision=lax.Precision.HIGHEST)
        out_ref[0] = jnp.where(half == 0, h[:, :16], h[:, 16:])

    return pl.pallas_call(
        body,
        grid=(2, _NB),
        in_specs=[
            pl.BlockSpec((1, 1, _BN), lambda h, i: (i, 0, 0)),
            pl.BlockSpec((64, 32), lambda h, i: (0, 0)),
        ],
        out_specs=pl.BlockSpec((1, _BN, 16), lambda h, i: (h, i, 0)),
        out_shape=jax.ShapeDtypeStruct((2, _NP, 16), jnp.float32),
    )(lab3, embp)


def _sc_agg(h_flat, src, dst, zz, dh):
    mesh = plsc.VectorSubcoreMesh(
        core_axis_name="c", subcore_axis_name="s",
        num_cores=_NC, num_subcores=_NS)

    @functools.partial(
        pl.kernel,
        out_type=jax.ShapeDtypeStruct((_NC * _NP, dh), jnp.float32),
        mesh=mesh,
        compiler_params=pltpu.CompilerParams(use_tc_tiling_on_sc=False),
        scratch_types=[
            pltpu.VMEM((1, _EB), jnp.int32),
            pltpu.VMEM((1, _EB), jnp.int32),
            pltpu.VMEM((_EB, dh), jnp.float32),
            pltpu.VMEM((_ZR, dh), jnp.float32),
            pltpu.VMEM_SHARED((_NP, dh), jnp.float32),
            pltpu.SemaphoreType.DMA,
        ],
    )
    def agg(h_hbm, src_hbm, dst_hbm, zz_hbm, out_hbm,
            src_v, dst_v, rows_v, zbuf, acc, sem):
        c = lax.axis_index("c")
        s = lax.axis_index("s")
        r0 = s * _RPT

        pltpu.sync_copy(zz_hbm, zbuf)

        def zbody(i, carry):
            pltpu.sync_copy(zbuf, acc.at[pl.ds(r0 + i * _ZR, _ZR)])
            return carry

        lax.fori_loop(0, _RPT // _ZR, zbody, 0)
        plsc.subcore_barrier()
        pl.delay(20000)

        e0 = s * _EPT

        def body(j, carry):
            eb = e0 + j * _EB
            pltpu.sync_copy(src_hbm.at[pl.ds(c * _E + eb, _EB)], src_v.at[0])
            pltpu.sync_copy(dst_hbm.at[pl.ds(eb, _EB)], dst_v.at[0])
            pltpu.async_copy(h_hbm.at[src_v.at[0]], rows_v, sem).wait()
            pltpu.sync_copy(rows_v, acc.at[dst_v.at[0]], add=True)
            return carry

        lax.fori_loop(0, _NBATCH, body, 0)
        plsc.subcore_barrier()
        pl.delay(20000)

        def wbody(i, carry):
            rr = r0 + i * _ZR
            pltpu.sync_copy(acc.at[pl.ds(rr, _ZR)], zbuf)
            pltpu.sync_copy(zbuf, out_hbm.at[pl.ds(c * _NP + rr, _ZR)])
            return carry

        lax.fori_loop(0, _RPT // _ZR, wbody, 0)

    return agg(h_flat, src, dst, zz)


def _mlp1_call(h0f, a1f, W11, b11, W12, b12):
    def body(ha, hb, aa, ab, w1, b1, w2, b2, out_ref):
        half = pl.program_id(0)
        x = jnp.concatenate([ha[...] + aa[...], hb[...] + ab[...]], axis=1)
        z = jnp.maximum(
            jnp.dot(x, w1[...], preferred_element_type=jnp.float32,
                    precision=lax.Precision.HIGHEST) + b1[...],
            0.0)
        z = jnp.dot(z, w2[...], preferred_element_type=jnp.float32,
                    precision=lax.Precision.HIGHEST) + b2[...]
        h1 = jnp.maximum(z, 0.0)
        out_ref[0] = jnp.where(half == 0, h1[:, :32], h1[:, 32:])

    lo = pl.BlockSpec((_BN, 16), lambda h, i: (i, 0))
    hi = pl.BlockSpec((_BN, 16), lambda h, i: (_NB + i, 0))
    full = lambda shp: pl.BlockSpec(shp, lambda h, i: tuple(0 for _ in shp))
    return pl.pallas_call(
        body,
        grid=(2, _NB),
        in_specs=[lo, hi, lo, hi,
                  full((32, 64)), full((1, 64)), full((64, 64)), full((1, 64))],
        out_specs=pl.BlockSpec((1, _BN, 32), lambda h, i: (h, i, 0)),
        out_shape=jax.ShapeDtypeStruct((2, _NP, 32), jnp.float32),
    )(h0f, h0f, a1f, a1f, W11, b11, W12, b12)


def _final_call(h1f, a2f, W21, b21, W22, b22, Ws1, bs1, Ws2, bs2):
    def body(ha, hb, aa, ab, w1, b1, w2, b2, ws1, bs1r, ws2, bs2r,
             out_ref, acc_ref):
        i = pl.program_id(0)
        x = jnp.concatenate([ha[...] + aa[...], hb[...] + ab[...]], axis=1)
        z = jnp.maximum(
            jnp.dot(x, w1[...], preferred_element_type=jnp.float32,
                    precision=lax.Precision.HIGHEST) + b1[...],
            0.0)
        z = jnp.dot(z, w2[...], preferred_element_type=jnp.float32,
                    precision=lax.Precision.HIGHEST) + b2[...]
        h2 = jnp.maximum(z, 0.0)
        rid = i * _BN + lax.broadcasted_iota(jnp.int32, (_BN, 64), 0)
        h2 = jnp.where(rid < _N, h2, 0.0)

        @pl.when(i == 0)
        def _():
            acc_ref[...] = jnp.zeros_like(acc_ref)

        acc_ref[...] += jnp.sum(h2, axis=0, keepdims=True)
        mean = acc_ref[...] * (1.0 / _N)
        t = jnp.maximum(
            jnp.dot(mean, ws1[...], preferred_element_type=jnp.float32,
                    precision=lax.Precision.HIGHEST)
            + bs1r[...], 0.0)
        out_ref[...] = (
            jnp.dot(t, ws2[...], preferred_element_type=jnp.float32,
                    precision=lax.Precision.HIGHEST)
            + bs2r[...])

    lo = pl.BlockSpec((_BN, 32), lambda i: (i, 0))
    hi = pl.BlockSpec((_BN, 32), lambda i: (_NB + i, 0))
    full = lambda shp: pl.BlockSpec(shp, lambda i: tuple(0 for _ in shp))
    return pl.pallas_call(
        body,
        grid=(_NB,),
        in_specs=[lo, hi, lo, hi,
                  full((64, 64)), full((1, 64)), full((64, 64)), full((1, 64)),
                  full((64, 64)), full((1, 64)), full((64, 1)), full((1, 1))],
        out_specs=pl.BlockSpec((1, 1), lambda i: (0, 0)),
        out_shape=jax.ShapeDtypeStruct((1, 1), jnp.float32),
        scratch_shapes=[pltpu.VMEM((1, 64), jnp.float32)],
    )(h1f, h1f, a2f, a2f, W21, b21, W22, b22, Ws1, bs1, Ws2, bs2)


def kernel(labels, edge_index, emb, W11, b11, W12, b12, W21, b21, W22, b22,
           Ws1, bs1, Ws2, bs2):
    labels = labels.astype(jnp.int32)
    src = edge_index[0].astype(jnp.int32)
    dst = edge_index[1].astype(jnp.int32)
    embp = jnp.zeros((64, 32), jnp.float32).at[:51].set(emb)
    labp = jnp.zeros((_NP,), jnp.int32).at[:_N].set(labels)
    lab3 = labp.reshape(_NB, 1, _BN)

    h0f = _embed_call(lab3, embp).reshape(_NC * _NP, 16)
    gsrc = jnp.concatenate([src, src + _NP])
    zz16 = jnp.zeros((_ZR, 16), jnp.float32)
    a1f = _sc_agg(h0f, gsrc, dst, zz16, 16)

    h1f = _mlp1_call(h0f, a1f, W11, b11.reshape(1, 64), W12,
                     b12.reshape(1, 64)).reshape(_NC * _NP, 32)
    zz32 = jnp.zeros((_ZR, 32), jnp.float32)
    a2f = _sc_agg(h1f, gsrc, dst, zz32, 32)

    out = _final_call(h1f, a2f, W21, b21.reshape(1, 64), W22,
                      b22.reshape(1, 64), Ws1, bs1.reshape(1, 64), Ws2,
                      bs2.reshape(1, 1))
    return out.reshape(1)


# speedup vs baseline: 5.7087x; 2.3197x over previous
import functools

import jax
import jax.numpy as jnp
from jax import lax
from jax.experimental import pallas as pl
from jax.experimental.pallas import tpu as pltpu
from jax.experimental.pallas import tpu_sc as plsc

_N = 50000
_NP = 51200
_E = 800000
_NC = 2
_NS = 16
_BN = 800
_NB = _NP // _BN
_EPT = _E // _NS
_EB = 80
_NBATCH = _EPT // _EB
_CH = 5
_NCHUNK = _NBATCH // _CH
_RPT = _NP // _NS
_ZR = 320


def _embed_call(lab3, embp):
    def body(lab_ref, emb_ref, out_ref):
        half = pl.program_id(0)
        lab = jnp.clip(lab_ref[0, 0, :], 0, 50)
        oh = (lab[:, None] == lax.broadcasted_iota(jnp.int32, (_BN, 64), 1))
        h = jnp.dot(oh.astype(jnp.float32), emb_ref[...],
                    preferred_element_type=jnp.float32,
                    precision=lax.Precision.HIGHEST)
        out_ref[0] = jnp.where(half == 0, h[:, :16], h[:, 16:])

    return pl.pallas_call(
        body,
        grid=(2, _NB),
        in_specs=[
            pl.BlockSpec((1, 1, _BN), lambda h, i: (i, 0, 0)),
            pl.BlockSpec((64, 32), lambda h, i: (0, 0)),
        ],
        out_specs=pl.BlockSpec((1, _BN, 16), lambda h, i: (h, i, 0)),
        out_shape=jax.ShapeDtypeStruct((2, _NP, 16), jnp.float32),
    )(lab3, embp)


def _sc_agg(h_flat, src, dst, zz, dh):
    mesh = plsc.VectorSubcoreMesh(
        core_axis_name="c", subcore_axis_name="s",
        num_cores=_NC, num_subcores=_NS)

    @functools.partial(
        pl.kernel,
        out_type=jax.ShapeDtypeStruct((_NC * _NP, dh), jnp.float32),
        mesh=mesh,
        compiler_params=pltpu.CompilerParams(use_tc_tiling_on_sc=False),
        scratch_types=[
            pltpu.VMEM((_CH, _EB), jnp.int32),
            pltpu.VMEM((_CH, _EB), jnp.int32),
            pltpu.VMEM((_CH, _EB, dh), jnp.float32),
            pltpu.VMEM((_ZR, dh), jnp.float32),
            pltpu.VMEM_SHARED((_NP, dh), jnp.float32),
            pltpu.SemaphoreType.DMA,
            pltpu.SemaphoreType.DMA,
        ],
    )
    def agg(h_hbm, src_hbm, dst_hbm, zz_hbm, out_hbm,
            src_v, dst_v, rows_v, zbuf, acc, gsem, ssem):
        c = lax.axis_index("c")
        s = lax.axis_index("s")
        r0 = s * _RPT

        pltpu.sync_copy(zz_hbm, zbuf)

        def zbody(i, carry):
            pltpu.sync_copy(zbuf, acc.at[pl.ds(r0 + i * _ZR, _ZR)])
            return carry

        lax.fori_loop(0, _RPT // _ZR, zbody, 0)
        plsc.subcore_barrier()
        pl.delay(20000)

        e0 = s * _EPT

        def body(j, carry):
            row0 = (e0 // _EB) + j * _CH
            i1 = pltpu.async_copy(
                src_hbm.at[pl.ds(c * (_E // _EB) + row0, _CH)], src_v, gsem)
            i2 = pltpu.async_copy(dst_hbm.at[pl.ds(row0, _CH)], dst_v, gsem)
            i1.wait()
            i2.wait()
            gathers = [
                pltpu.async_copy(h_hbm.at[src_v.at[b]], rows_v.at[b], gsem)
                for b in range(_CH)
            ]
            scatters = []
            for b in range(_CH):
                gathers[b].wait()
                scatters.append(
                    pltpu.async_copy(rows_v.at[b], acc.at[dst_v.at[b]], ssem,
                                     add=True))
            for d in scatters:
                d.wait()
            return carry

        lax.fori_loop(0, _NCHUNK, body, 0)
        plsc.subcore_barrier()
        pl.delay(20000)

        def wbody(i, carry):
            rr = r0 + i * _ZR
            pltpu.sync_copy(acc.at[pl.ds(rr, _ZR)], zbuf)
            pltpu.sync_copy(zbuf, out_hbm.at[pl.ds(c * _NP + rr, _ZR)])
            return carry

        lax.fori_loop(0, _RPT // _ZR, wbody, 0)

    return agg(h_flat, src, dst, zz)


def _mlp1_call(h0f, a1f, W11, b11, W12, b12):
    def body(ha, hb, aa, ab, w1, b1, w2, b2, out_ref):
        half = pl.program_id(0)
        x = jnp.concatenate([ha[...] + aa[...], hb[...] + ab[...]], axis=1)
        z = jnp.maximum(
            jnp.dot(x, w1[...], preferred_element_type=jnp.float32,
                    precision=lax.Precision.HIGHEST) + b1[...],
            0.0)
        z = jnp.dot(z, w2[...], preferred_element_type=jnp.float32,
                    precision=lax.Precision.HIGHEST) + b2[...]
        h1 = jnp.maximum(z, 0.0)
        out_ref[0] = jnp.where(half == 0, h1[:, :32], h1[:, 32:])

    lo = pl.BlockSpec((_BN, 16), lambda h, i: (i, 0))
    hi = pl.BlockSpec((_BN, 16), lambda h, i: (_NB + i, 0))
    full = lambda shp: pl.BlockSpec(shp, lambda h, i: tuple(0 for _ in shp))
    return pl.pallas_call(
        body,
        grid=(2, _NB),
        in_specs=[lo, hi, lo, hi,
                  full((32, 64)), full((1, 64)), full((64, 64)), full((1, 64))],
        out_specs=pl.BlockSpec((1, _BN, 32), lambda h, i: (h, i, 0)),
        out_shape=jax.ShapeDtypeStruct((2, _NP, 32), jnp.float32),
    )(h0f, h0f, a1f, a1f, W11, b11, W12, b12)


def _final_call(h1f, a2f, W21, b21, W22, b22, Ws1, bs1, Ws2, bs2):
    def body(ha, hb, aa, ab, w1, b1, w2, b2, ws1, bs1r, ws2, bs2r,
             out_ref, acc_ref):
        i = pl.program_id(0)
        x = jnp.concatenate([ha[...] + aa[...], hb[...] + ab[...]], axis=1)
        z = jnp.maximum(
            jnp.dot(x, w1[...], preferred_element_type=jnp.float32,
                    precision=lax.Precision.HIGHEST) + b1[...],
            0.0)
        z = jnp.dot(z, w2[...], preferred_element_type=jnp.float32,
                    precision=lax.Precision.HIGHEST) + b2[...]
        h2 = jnp.maximum(z, 0.0)
        rid = i * _BN + lax.broadcasted_iota(jnp.int32, (_BN, 64), 0)
        h2 = jnp.where(rid < _N, h2, 0.0)

        @pl.when(i == 0)
        def _():
            acc_ref[...] = jnp.zeros_like(acc_ref)

        acc_ref[...] += jnp.sum(h2, axis=0, keepdims=True)
        mean = acc_ref[...] * (1.0 / _N)
        t = jnp.maximum(
            jnp.dot(mean, ws1[...], preferred_element_type=jnp.float32,
                    precision=lax.Precision.HIGHEST)
            + bs1r[...], 0.0)
        out_ref[...] = (
            jnp.dot(t, ws2[...], preferred_element_type=jnp.float32,
                    precision=lax.Precision.HIGHEST)
            + bs2r[...])

    lo = pl.BlockSpec((_BN, 32), lambda i: (i, 0))
    hi = pl.BlockSpec((_BN, 32), lambda i: (_NB + i, 0))
    full = lambda shp: pl.BlockSpec(shp, lambda i: tuple(0 for _ in shp))
    return pl.pallas_call(
        body,
        grid=(_NB,),
        in_specs=[lo, hi, lo, hi,
                  full((64, 64)), full((1, 64)), full((64, 64)), full((1, 64)),
                  full((64, 64)), full((1, 64)), full((64, 1)), full((1, 1))],
        out_specs=pl.BlockSpec((1, 1), lambda i: (0, 0)),
        out_shape=jax.ShapeDtypeStruct((1, 1), jnp.float32),
        scratch_shapes=[pltpu.VMEM((1, 64), jnp.float32)],
    )(h1f, h1f, a2f, a2f, W21, b21, W22, b22, Ws1, bs1, Ws2, bs2)


def kernel(labels, edge_index, emb, W11, b11, W12, b12, W21, b21, W22, b22,
           Ws1, bs1, Ws2, bs2):
    labels = labels.astype(jnp.int32)
    src = edge_index[0].astype(jnp.int32)
    dst = edge_index[1].astype(jnp.int32)
    embp = jnp.zeros((64, 32), jnp.float32).at[:51].set(emb)
    labp = jnp.zeros((_NP,), jnp.int32).at[:_N].set(labels)
    lab3 = labp.reshape(_NB, 1, _BN)

    h0f = _embed_call(lab3, embp).reshape(_NC * _NP, 16)
    gsrc = jnp.concatenate([src, src + _NP]).reshape(2 * _E // _EB, _EB)
    dst2 = dst.reshape(_E // _EB, _EB)
    zz16 = jnp.zeros((_ZR, 16), jnp.float32)
    a1f = _sc_agg(h0f, gsrc, dst2, zz16, 16)

    h1f = _mlp1_call(h0f, a1f, W11, b11.reshape(1, 64), W12,
                     b12.reshape(1, 64)).reshape(_NC * _NP, 32)
    zz32 = jnp.zeros((_ZR, 32), jnp.float32)
    a2f = _sc_agg(h1f, gsrc, dst2, zz32, 32)

    out = _final_call(h1f, a2f, W21, b21.reshape(1, 64), W22,
                      b22.reshape(1, 64), Ws1, bs1.reshape(1, 64), Ws2,
                      bs2.reshape(1, 1))
    return out.reshape(1)


# speedup vs baseline: 5.7365x; 1.0049x over previous
import functools

import jax
import jax.numpy as jnp
from jax import lax
from jax.experimental import pallas as pl
from jax.experimental.pallas import tpu as pltpu
from jax.experimental.pallas import tpu_sc as plsc

_N = 50000
_NP = 51200
_E = 800000
_NC = 2
_NS = 16
_BN = 800
_NB = _NP // _BN
_EPT = _E // _NS
_EB = 80
_NBATCH = _EPT // _EB
_CH = 5
_NCHUNK = _NBATCH // _CH
_RPT = _NP // _NS
_ZR = 320


def _embed_call(lab3, embp):
    def body(lab_ref, emb_ref, out_ref):
        half = pl.program_id(0)
        lab = jnp.clip(lab_ref[0, 0, :], 0, 50)
        oh = (lab[:, None] == lax.broadcasted_iota(jnp.int32, (_BN, 64), 1))
        h = jnp.dot(oh.astype(jnp.float32), emb_ref[...],
                    preferred_element_type=jnp.float32,
                    precision=lax.Precision.HIGHEST)
        out_ref[0] = jnp.where(half == 0, h[:, :16], h[:, 16:])

    return pl.pallas_call(
        body,
        grid=(2, _NB),
        in_specs=[
            pl.BlockSpec((1, 1, _BN), lambda h, i: (i, 0, 0)),
            pl.BlockSpec((64, 32), lambda h, i: (0, 0)),
        ],
        out_specs=pl.BlockSpec((1, _BN, 16), lambda h, i: (h, i, 0)),
        out_shape=jax.ShapeDtypeStruct((2, _NP, 16), jnp.float32),
    )(lab3, embp)


def _sc_agg(h_flat, src, dst, zz, dh):
    mesh = plsc.VectorSubcoreMesh(
        core_axis_name="c", subcore_axis_name="s",
        num_cores=_NC, num_subcores=_NS)

    @functools.partial(
        pl.kernel,
        out_type=jax.ShapeDtypeStruct((_NC * _NP, dh), jnp.float32),
        mesh=mesh,
        compiler_params=pltpu.CompilerParams(use_tc_tiling_on_sc=False),
        scratch_types=[
            pltpu.VMEM((_CH, _EB), jnp.int32),
            pltpu.VMEM((_CH, _EB), jnp.int32),
            pltpu.VMEM((_CH, _EB, dh), jnp.float32),
            pltpu.VMEM((_ZR, dh), jnp.float32),
            pltpu.VMEM_SHARED((_NP, dh), jnp.float32),
            pltpu.SemaphoreType.DMA((_CH,)),
            pltpu.SemaphoreType.DMA((_CH,)),
        ],
    )
    def agg(h_hbm, src_hbm, dst_hbm, zz_hbm, out_hbm,
            src_v, dst_v, rows_v, zbuf, acc, gsem, ssem):
        c = lax.axis_index("c")
        s = lax.axis_index("s")
        r0 = s * _RPT

        pltpu.sync_copy(zz_hbm, zbuf)

        def zbody(i, carry):
            pltpu.sync_copy(zbuf, acc.at[pl.ds(r0 + i * _ZR, _ZR)])
            return carry

        lax.fori_loop(0, _RPT // _ZR, zbody, 0)
        plsc.subcore_barrier()
        pl.delay(20000)

        e0 = s * _EPT

        def body(j, carry):
            row0 = (e0 // _EB) + j * _CH
            i1 = pltpu.async_copy(
                src_hbm.at[pl.ds(c * (_E // _EB) + row0, _CH)], src_v,
                gsem.at[0])
            i2 = pltpu.async_copy(dst_hbm.at[pl.ds(row0, _CH)], dst_v,
                                  gsem.at[1])
            i1.wait()
            i2.wait()
            gathers = [
                pltpu.async_copy(h_hbm.at[src_v.at[b]], rows_v.at[b],
                                 gsem.at[b])
                for b in range(_CH)
            ]
            scatters = []
            for b in range(_CH):
                gathers[b].wait()
                scatters.append(
                    pltpu.async_copy(rows_v.at[b], acc.at[dst_v.at[b]],
                                     ssem.at[b], add=True))
            for d in scatters:
                d.wait()
            return carry

        lax.fori_loop(0, _NCHUNK, body, 0)
        plsc.subcore_barrier()
        pl.delay(20000)

        def wbody(i, carry):
            rr = r0 + i * _ZR
            pltpu.sync_copy(acc.at[pl.ds(rr, _ZR)], zbuf)
            pltpu.sync_copy(zbuf, out_hbm.at[pl.ds(c * _NP + rr, _ZR)])
            return carry

        lax.fori_loop(0, _RPT // _ZR, wbody, 0)

    return agg(h_flat, src, dst, zz)


def _mlp1_call(h0f, a1f, W11, b11, W12, b12):
    def body(ha, hb, aa, ab, w1, b1, w2, b2, out_ref):
        half = pl.program_id(0)
        x = jnp.concatenate([ha[...] + aa[...], hb[...] + ab[...]], axis=1)
        z = jnp.maximum(
            jnp.dot(x, w1[...], preferred_element_type=jnp.float32,
                    precision=lax.Precision.HIGHEST) + b1[...],
            0.0)
        z = jnp.dot(z, w2[...], preferred_element_type=jnp.float32,
                    precision=lax.Precision.HIGHEST) + b2[...]
        h1 = jnp.maximum(z, 0.0)
        out_ref[0] = jnp.where(half == 0, h1[:, :32], h1[:, 32:])

    lo = pl.BlockSpec((_BN, 16), lambda h, i: (i, 0))
    hi = pl.BlockSpec((_BN, 16), lambda h, i: (_NB + i, 0))
    full = lambda shp: pl.BlockSpec(shp, lambda h, i: tuple(0 for _ in shp))
    return pl.pallas_call(
        body,
        grid=(2, _NB),
        in_specs=[lo, hi, lo, hi,
                  full((32, 64)), full((1, 64)), full((64, 64)), full((1, 64))],
        out_specs=pl.BlockSpec((1, _BN, 32), lambda h, i: (h, i, 0)),
        out_shape=jax.ShapeDtypeStruct((2, _NP, 32), jnp.float32),
    )(h0f, h0f, a1f, a1f, W11, b11, W12, b12)


def _final_call(h1f, a2f, W21, b21, W22, b22, Ws1, bs1, Ws2, bs2):
    def body(ha, hb, aa, ab, w1, b1, w2, b2, ws1, bs1r, ws2, bs2r,
             out_ref, acc_ref):
        i = pl.program_id(0)
        x = jnp.concatenate([ha[...] + aa[...], hb[...] + ab[...]], axis=1)
        z = jnp.maximum(
            jnp.dot(x, w1[...], preferred_element_type=jnp.float32,
                    precision=lax.Precision.HIGHEST) + b1[...],
            0.0)
        z = jnp.dot(z, w2[...], preferred_element_type=jnp.float32,
                    precision=lax.Precision.HIGHEST) + b2[...]
        h2 = jnp.maximum(z, 0.0)
        rid = i * _BN + lax.broadcasted_iota(jnp.int32, (_BN, 64), 0)
        h2 = jnp.where(rid < _N, h2, 0.0)

        @pl.when(i == 0)
        def _():
            acc_ref[...] = jnp.zeros_like(acc_ref)

        acc_ref[...] += jnp.sum(h2, axis=0, keepdims=True)
        mean = acc_ref[...] * (1.0 / _N)
        t = jnp.maximum(
            jnp.dot(mean, ws1[...], preferred_element_type=jnp.float32,
                    precision=lax.Precision.HIGHEST)
            + bs1r[...], 0.0)
        out_ref[...] = (
            jnp.dot(t, ws2[...], preferred_element_type=jnp.float32,
                    precision=lax.Precision.HIGHEST)
            + bs2r[...])

    lo = pl.BlockSpec((_BN, 32), lambda i: (i, 0))
    hi = pl.BlockSpec((_BN, 32), lambda i: (_NB + i, 0))
    full = lambda shp: pl.BlockSpec(shp, lambda i: tuple(0 for _ in shp))
    return pl.pallas_call(
        body,
        grid=(_NB,),
        in_specs=[lo, hi, lo, hi,
                  full((64, 64)), full((1, 64)), full((64, 64)), full((1, 64)),
                  full((64, 64)), full((1, 64)), full((64, 1)), full((1, 1))],
        out_specs=pl.BlockSpec((1, 1), lambda i: (0, 0)),
        out_shape=jax.ShapeDtypeStruct((1, 1), jnp.float32),
        scratch_shapes=[pltpu.VMEM((1, 64), jnp.float32)],
    )(h1f, h1f, a2f, a2f, W21, b21, W22, b22, Ws1, bs1, Ws2, bs2)


def kernel(labels, edge_index, emb, W11, b11, W12, b12, W21, b21, W22, b22,
           Ws1, bs1, Ws2, bs2):
    labels = labels.astype(jnp.int32)
    src = edge_index[0].astype(jnp.int32)
    dst = edge_index[1].astype(jnp.int32)
    embp = jnp.zeros((64, 32), jnp.float32).at[:51].set(emb)
    labp = jnp.zeros((_NP,), jnp.int32).at[:_N].set(labels)
    lab3 = labp.reshape(_NB, 1, _BN)

    h0f = _embed_call(lab3, embp).reshape(_NC * _NP, 16)
    gsrc = jnp.concatenate([src, src + _NP]).reshape(2 * _E // _EB, _EB)
    dst2 = dst.reshape(_E // _EB, _EB)
    zz16 = jnp.zeros((_ZR, 16), jnp.float32)
    a1f = _sc_agg(h0f, gsrc, dst2, zz16, 16)

    h1f = _mlp1_call(h0f, a1f, W11, b11.reshape(1, 64), W12,
                     b12.reshape(1, 64)).reshape(_NC * _NP, 32)
    zz32 = jnp.zeros((_ZR, 32), jnp.float32)
    a2f = _sc_agg(h1f, gsrc, dst2, zz32, 32)

    out = _final_call(h1f, a2f, W21, b21.reshape(1, 64), W22,
                      b22.reshape(1, 64), Ws1, bs1.reshape(1, 64), Ws2,
                      bs2.reshape(1, 1))
    return out.reshape(1)


# speedup vs baseline: 6.5259x; 1.1376x over previous
import functools

import jax
import jax.numpy as jnp
from jax import lax
from jax.experimental import pallas as pl
from jax.experimental.pallas import tpu as pltpu
from jax.experimental.pallas import tpu_sc as plsc

_N = 50000
_NP = 51200
_E = 800000
_NC = 2
_NS = 16
_BN = 800
_NB = _NP // _BN
_EPT = _E // _NS
_EB = 80
_NBATCH = _EPT // _EB
_CH = 5
_SUB = 5
_SC_BATCH = _CH * _SUB
_NCHUNK = _NBATCH // _SC_BATCH
_RPT = _NP // _NS
_ZR = 320


def _embed_call(lab3, embp):
    def body(lab_ref, emb_ref, out_ref):
        half = pl.program_id(0)
        lab = jnp.clip(lab_ref[0, 0, :], 0, 50)
        oh = (lab[:, None] == lax.broadcasted_iota(jnp.int32, (_BN, 64), 1))
        h = jnp.dot(oh.astype(jnp.float32), emb_ref[...],
                    preferred_element_type=jnp.float32,
                    precision=lax.Precision.HIGHEST)
        out_ref[0] = jnp.where(half == 0, h[:, :16], h[:, 16:])

    return pl.pallas_call(
        body,
        grid=(2, _NB),
        in_specs=[
            pl.BlockSpec((1, 1, _BN), lambda h, i: (i, 0, 0)),
            pl.BlockSpec((64, 32), lambda h, i: (0, 0)),
        ],
        out_specs=pl.BlockSpec((1, _BN, 16), lambda h, i: (h, i, 0)),
        out_shape=jax.ShapeDtypeStruct((2, _NP, 16), jnp.float32),
    )(lab3, embp)


def _sc_agg(h_flat, src, dst, zz, dh):
    mesh = plsc.VectorSubcoreMesh(
        core_axis_name="c", subcore_axis_name="s",
        num_cores=_NC, num_subcores=_NS)

    @functools.partial(
        pl.kernel,
        out_type=jax.ShapeDtypeStruct((_NC * _NP, dh), jnp.float32),
        mesh=mesh,
        compiler_params=pltpu.CompilerParams(use_tc_tiling_on_sc=False),
        scratch_types=[
            pltpu.VMEM((_SC_BATCH, _EB), jnp.int32),
            pltpu.VMEM((_SC_BATCH, _EB), jnp.int32),
            pltpu.VMEM((_CH, _EB, dh), jnp.float32),
            pltpu.VMEM((_ZR, dh), jnp.float32),
            pltpu.VMEM_SHARED((_NP, dh), jnp.float32),
            pltpu.SemaphoreType.DMA((_CH,)),
            pltpu.SemaphoreType.DMA((_CH,)),
        ],
    )
    def agg(h_hbm, src_hbm, dst_hbm, zz_hbm, out_hbm,
            src_v, dst_v, rows_v, zbuf, acc, gsem, ssem):
        c = lax.axis_index("c")
        s = lax.axis_index("s")
        r0 = s * _RPT

        pltpu.sync_copy(zz_hbm, zbuf)

        def zbody(i, carry):
            pltpu.sync_copy(zbuf, acc.at[pl.ds(r0 + i * _ZR, _ZR)])
            return carry

        lax.fori_loop(0, _RPT // _ZR, zbody, 0)
        plsc.subcore_barrier()
        pl.delay(20000)

        e0 = s * _EPT

        def body(j, carry):
            row0 = (e0 // _EB) + j * _SC_BATCH
            i1 = pltpu.async_copy(
                src_hbm.at[pl.ds(c * (_E // _EB) + row0, _SC_BATCH)], src_v,
                gsem.at[0])
            i2 = pltpu.async_copy(dst_hbm.at[pl.ds(row0, _SC_BATCH)], dst_v,
                                  gsem.at[1])
            i1.wait()
            i2.wait()
            prev = []
            for sub in range(_SUB):
                gathers = []
                for b in range(_CH):
                    if prev:
                        prev[b].wait()
                    gathers.append(pltpu.async_copy(
                        h_hbm.at[src_v.at[sub * _CH + b]], rows_v.at[b],
                        gsem.at[b]))
                nxt = []
                for b in range(_CH):
                    gathers[b].wait()
                    nxt.append(pltpu.async_copy(
                        rows_v.at[b], acc.at[dst_v.at[sub * _CH + b]],
                        ssem.at[b], add=True))
                prev = nxt
            for d in prev:
                d.wait()
            return carry

        lax.fori_loop(0, _NCHUNK, body, 0)
        plsc.subcore_barrier()
        pl.delay(20000)

        def wbody(i, carry):
            rr = r0 + i * _ZR
            pltpu.sync_copy(acc.at[pl.ds(rr, _ZR)], zbuf)
            pltpu.sync_copy(zbuf, out_hbm.at[pl.ds(c * _NP + rr, _ZR)])
            return carry

        lax.fori_loop(0, _RPT // _ZR, wbody, 0)

    return agg(h_flat, src, dst, zz)


def _mlp1_call(h0f, a1f, W11, b11, W12, b12):
    def body(ha, hb, aa, ab, w1, b1, w2, b2, out_ref):
        half = pl.program_id(0)
        x = jnp.concatenate([ha[...] + aa[...], hb[...] + ab[...]], axis=1)
        z = jnp.maximum(
            jnp.dot(x, w1[...], preferred_element_type=jnp.float32,
                    precision=lax.Precision.HIGHEST) + b1[...],
            0.0)
        z = jnp.dot(z, w2[...], preferred_element_type=jnp.float32,
                    precision=lax.Precision.HIGHEST) + b2[...]
        h1 = jnp.maximum(z, 0.0)
        out_ref[0] = jnp.where(half == 0, h1[:, :32], h1[:, 32:])

    lo = pl.BlockSpec((_BN, 16), lambda h, i: (i, 0))
    hi = pl.BlockSpec((_BN, 16), lambda h, i: (_NB + i, 0))
    full = lambda shp: pl.BlockSpec(shp, lambda h, i: tuple(0 for _ in shp))
    return pl.pallas_call(
        body,
        grid=(2, _NB),
        in_specs=[lo, hi, lo, hi,
                  full((32, 64)), full((1, 64)), full((64, 64)), full((1, 64))],
        out_specs=pl.BlockSpec((1, _BN, 32), lambda h, i: (h, i, 0)),
        out_shape=jax.ShapeDtypeStruct((2, _NP, 32), jnp.float32),
    )(h0f, h0f, a1f, a1f, W11, b11, W12, b12)


def _final_call(h1f, a2f, W21, b21, W22, b22, Ws1, bs1, Ws2, bs2):
    def body(ha, hb, aa, ab, w1, b1, w2, b2, ws1, bs1r, ws2, bs2r,
             out_ref, acc_ref):
        i = pl.program_id(0)
        x = jnp.concatenate([ha[...] + aa[...], hb[...] + ab[...]], axis=1)
        z = jnp.maximum(
            jnp.dot(x, w1[...], preferred_element_type=jnp.float32,
                    precision=lax.Precision.HIGHEST) + b1[...],
            0.0)
        z = jnp.dot(z, w2[...], preferred_element_type=jnp.float32,
                    precision=lax.Precision.HIGHEST) + b2[...]
        h2 = jnp.maximum(z, 0.0)
        rid = i * _BN + lax.broadcasted_iota(jnp.int32, (_BN, 64), 0)
        h2 = jnp.where(rid < _N, h2, 0.0)

        @pl.when(i == 0)
        def _():
            acc_ref[...] = jnp.zeros_like(acc_ref)

        acc_ref[...] += jnp.sum(h2, axis=0, keepdims=True)
        mean = acc_ref[...] * (1.0 / _N)
        t = jnp.maximum(
            jnp.dot(mean, ws1[...], preferred_element_type=jnp.float32,
                    precision=lax.Precision.HIGHEST)
            + bs1r[...], 0.0)
        out_ref[...] = (
            jnp.dot(t, ws2[...], preferred_element_type=jnp.float32,
                    precision=lax.Precision.HIGHEST)
            + bs2r[...])

    lo = pl.BlockSpec((_BN, 32), lambda i: (i, 0))
    hi = pl.BlockSpec((_BN, 32), lambda i: (_NB + i, 0))
    full = lambda shp: pl.BlockSpec(shp, lambda i: tuple(0 for _ in shp))
    return pl.pallas_call(
        body,
        grid=(_NB,),
        in_specs=[lo, hi, lo, hi,
                  full((64, 64)), full((1, 64)), full((64, 64)), full((1, 64)),
                  full((64, 64)), full((1, 64)), full((64, 1)), full((1, 1))],
        out_specs=pl.BlockSpec((1, 1), lambda i: (0, 0)),
        out_shape=jax.ShapeDtypeStruct((1, 1), jnp.float32),
        scratch_shapes=[pltpu.VMEM((1, 64), jnp.float32)],
    )(h1f, h1f, a2f, a2f, W21, b21, W22, b22, Ws1, bs1, Ws2, bs2)


def kernel(labels, edge_index, emb, W11, b11, W12, b12, W21, b21, W22, b22,
           Ws1, bs1, Ws2, bs2):
    labels = labels.astype(jnp.int32)
    src = edge_index[0].astype(jnp.int32)
    dst = edge_index[1].astype(jnp.int32)
    embp = jnp.zeros((64, 32), jnp.float32).at[:51].set(emb)
    labp = jnp.zeros((_NP,), jnp.int32).at[:_N].set(labels)
    lab3 = labp.reshape(_NB, 1, _BN)

    h0f = _embed_call(lab3, embp).reshape(_NC * _NP, 16)
    gsrc = jnp.concatenate([src, src + _NP]).reshape(2 * _E // _EB, _EB)
    dst2 = dst.reshape(_E // _EB, _EB)
    zz16 = jnp.zeros((_ZR, 16), jnp.float32)
    a1f = _sc_agg(h0f, gsrc, dst2, zz16, 16)

    h1f = _mlp1_call(h0f, a1f, W11, b11.reshape(1, 64), W12,
                     b12.reshape(1, 64)).reshape(_NC * _NP, 32)
    zz32 = jnp.zeros((_ZR, 32), jnp.float32)
    a2f = _sc_agg(h1f, gsrc, dst2, zz32, 32)

    out = _final_call(h1f, a2f, W21, b21.reshape(1, 64), W22,
                      b22.reshape(1, 64), Ws1, bs1.reshape(1, 64), Ws2,
                      bs2.reshape(1, 1))
    return out.reshape(1)
